# baseline (speedup 1.0000x reference)
import jax
import jax.numpy as jnp
from jax import lax
from jax.experimental import pallas as pl
from jax.experimental.pallas import tpu as pltpu
from jax.experimental.pallas import tpu_sc as plsc

WORD = 1000000
D = 64
B = 16384
NNEG = 5

NC = 2
NS = 16
NW = NC * NS
BPW = B // NW
CHUNK = 32
NCHUNK = BPW // CHUNK
DV = D // 16


def _sc_body(u_hbm, v_hbm,
             urow_hbm, vrow_hbm, nrow_hbm, upar_hbm, vpar_hbm, npar_hbm,
             out1_hbm, out2_hbm,
             urow, vrow, nrow, upar, vpar, npar,
             ubuf0, vbuf0, nbuf00, nbuf01, nbuf02, nbuf03, nbuf04,
             ubuf1, vbuf1, nbuf10, nbuf11, nbuf12, nbuf13, nbuf14,
             accbuf, sem0, sem1):
    wid = lax.axis_index("s") * NC + lax.axis_index("c")
    bufs = [
        (ubuf0, vbuf0, [nbuf00, nbuf01, nbuf02, nbuf03, nbuf04], sem0),
        (ubuf1, vbuf1, [nbuf10, nbuf11, nbuf12, nbuf13, nbuf14], sem1),
    ]

    pltpu.sync_copy(urow_hbm.at[wid], urow)
    pltpu.sync_copy(vrow_hbm.at[wid], vrow)
    pltpu.sync_copy(nrow_hbm.at[wid], nrow)
    pltpu.sync_copy(upar_hbm.at[wid], upar)
    pltpu.sync_copy(vpar_hbm.at[wid], vpar)
    pltpu.sync_copy(npar_hbm.at[wid], npar)

    def fire(c, p):
        ub, vb, nb, sem = bufs[p]
        pltpu.async_copy(u_hbm.at[urow.at[c]], ub, sem)
        pltpu.async_copy(v_hbm.at[vrow.at[c]], vb, sem)
        for n in range(NNEG):
            pltpu.async_copy(v_hbm.at[nrow.at[n, c]], nb[n], sem)

    def drain(p):
        ub, vb, nb, sem = bufs[p]
        pltpu.make_async_copy(u_hbm.at[pl.ds(0, CHUNK)], ub, sem).wait()
        pltpu.make_async_copy(v_hbm.at[pl.ds(0, CHUNK)], vb, sem).wait()
        for n in range(NNEG):
            pltpu.make_async_copy(v_hbm.at[pl.ds(0, CHUNK)], nb[n], sem).wait()

    def dot_pass(c, ub, pb, pbpar_slice, acc):
        def body(t, a):
            base = t * 16
            pu = upar[c, pl.ds(base, 16)]
            pp = pbpar_slice(pl.ds(base, 16))
            for l in range(16):
                r = base + l
                ou = pu[l] * 64
                op = pp[l] * 64
                for q in range(DV):
                    a = a + (ub[r, pl.ds(ou + 16 * q, 16)]
                             * pb[r, pl.ds(op + 16 * q, 16)])
            return a

        return lax.fori_loop(0, CHUNK // 16, body, acc)

    def compute(c, p, accs):
        ub, vb, nb, _ = bufs[p]
        a1, a2 = accs
        a1 = dot_pass(c, ub, vb, lambda s: vpar[c, s], a1)
        for n in range(NNEG):
            a2 = dot_pass(c, ub, nb[n], lambda s, n=n: npar[n, c, s], a2)
        return (a1, a2)

    z = jnp.zeros((16,), jnp.float32)
    fire(0, 0)
    fire(1, 1)

    def chunk_pair(g, accs):
        c0 = 2 * g
        drain(0)
        accs = compute(c0, 0, accs)

        @pl.when(c0 + 2 < NCHUNK)
        def _():
            fire(c0 + 2, 0)

        drain(1)
        accs = compute(c0 + 1, 1, accs)

        @pl.when(c0 + 3 < NCHUNK)
        def _():
            fire(c0 + 3, 1)

        return accs

    accs = lax.fori_loop(0, NCHUNK // 2, chunk_pair, (z, z))

    accbuf[...] = accs[0]
    pltpu.sync_copy(accbuf, out1_hbm.at[wid])
    accbuf[...] = accs[1]
    pltpu.sync_copy(accbuf, out2_hbm.at[wid])


@jax.jit
def _skipgram(u_table, v_table, urow_w, vrow_w, nrow_w, upar_w, vpar_w, npar_w):
    u2 = u_table.reshape(WORD // 2, 2 * D)
    v2 = v_table.reshape(WORD // 2, 2 * D)
    mesh = plsc.VectorSubcoreMesh(core_axis_name="c", subcore_axis_name="s")
    row = pltpu.VMEM((CHUNK, 2 * D), jnp.float32)
    f = pl.kernel(
        _sc_body,
        out_type=(
            jax.ShapeDtypeStruct((NW, 16), jnp.float32),
            jax.ShapeDtypeStruct((NW, 16), jnp.float32),
        ),
        mesh=mesh,
        scratch_types=[
            pltpu.VMEM((NCHUNK, CHUNK), jnp.int32),
            pltpu.VMEM((NCHUNK, CHUNK), jnp.int32),
            pltpu.VMEM((NNEG, NCHUNK, CHUNK), jnp.int32),
            pltpu.VMEM((NCHUNK, CHUNK), jnp.int32),
            pltpu.VMEM((NCHUNK, CHUNK), jnp.int32),
            pltpu.VMEM((NNEG, NCHUNK, CHUNK), jnp.int32),
            row, row, row, row, row, row, row,
            row, row, row, row, row, row, row,
            pltpu.VMEM((16,), jnp.float32),
            pltpu.SemaphoreType.DMA,
            pltpu.SemaphoreType.DMA,
        ],
    )
    out1, out2 = f(u2, v2, urow_w, vrow_w, nrow_w, upar_w, vpar_w, npar_w)
    s1 = jnp.sum(out1)
    s2 = jnp.sum(out2)
    return -(jax.nn.log_sigmoid(s1) + jax.nn.log_sigmoid(-s2))


def kernel(u_table, v_table, pos_u, pos_v, neg_v):
    urow_w = (pos_u >> 1).reshape(NW, NCHUNK, CHUNK)
    upar_w = (pos_u & 1).reshape(NW, NCHUNK, CHUNK)
    vrow_w = (pos_v >> 1).reshape(NW, NCHUNK, CHUNK)
    vpar_w = (pos_v & 1).reshape(NW, NCHUNK, CHUNK)
    neg_t = neg_v.reshape(NW, NCHUNK, CHUNK, NNEG).transpose(0, 3, 1, 2)
    nrow_w = neg_t >> 1
    npar_w = neg_t & 1
    return _skipgram(u_table, v_table, urow_w, vrow_w, nrow_w,
                     upar_w, vpar_w, npar_w)
